# baseline (speedup 1.0000x reference)
import functools

import jax
import jax.numpy as jnp
from jax import lax
from jax.experimental import pallas as pl
from jax.experimental.pallas import tpu as pltpu
from jax.experimental.pallas import tpu_sc as plsc


def _make_gather(V, D, B):
    info = plsc.get_sparse_core_info()
    NC, NS = info.num_cores, info.num_subcores
    NW = NC * NS
    assert B % NW == 0
    b_per_w = B // NW
    C = 80
    assert b_per_w % C == 0
    steps = b_per_w // C

    mesh = plsc.VectorSubcoreMesh(core_axis_name="c", subcore_axis_name="s")

    NBUF = 8
    assert steps % NBUF == 0
    rounds = steps // NBUF

    @functools.partial(
        pl.kernel,
        mesh=mesh,
        out_type=jax.ShapeDtypeStruct((B, D), jnp.float32),
        scratch_types=[
            pltpu.VMEM((b_per_w,), jnp.int32),
            pltpu.VMEM((NBUF, C, D), jnp.float32),
            [pltpu.SemaphoreType.DMA] * NBUF,
            [pltpu.SemaphoreType.DMA] * NBUF,
        ],
    )
    def k(table_hbm, idx_hbm, out_hbm, idx_v, rows, gsems, osems):
        wid = lax.axis_index("s") * NC + lax.axis_index("c")
        base = wid * b_per_w
        pltpu.sync_copy(idx_hbm.at[pl.ds(base, b_per_w)], idx_v)

        def gather(g, b):
            return pltpu.make_async_copy(
                table_hbm.at[idx_v.at[pl.ds(g * C, C)]], rows.at[b], gsems[b]
            )

        def put(g, b):
            return pltpu.make_async_copy(
                rows.at[b], out_hbm.at[pl.ds(base + g * C, C)], osems[b]
            )

        for b in range(NBUF):
            gather(b, b).start()

        def step(i, carry):
            g0 = i * NBUF
            for b in range(NBUF):
                gather(g0 + b, b).wait()
                put(g0 + b, b).start()
            for b in range(NBUF):
                put(g0 + b, b).wait()
                gather(g0 + NBUF + b, b).start()
            return carry

        lax.fori_loop(0, rounds - 1, step, 0)

        g0 = (rounds - 1) * NBUF
        for b in range(NBUF):
            gather(g0 + b, b).wait()
            put(g0 + b, b).start()
        for b in range(NBUF):
            put(g0 + b, b).wait()

    return k


_GATHER_CACHE = {}


def kernel(input_ids, word_embeddings):
    V, D = word_embeddings.shape
    shape = input_ids.shape
    B = 1
    for s in shape:
        B *= s
    key = (V, D, B)
    if key not in _GATHER_CACHE:
        _GATHER_CACHE[key] = _make_gather(V, D, B)
    flat = input_ids.reshape(B).astype(jnp.int32)
    out = _GATHER_CACHE[key](word_embeddings, flat)
    return out.reshape(shape + (D,))


# speedup vs baseline: 1.0141x; 1.0141x over previous
import functools

import jax
import jax.numpy as jnp
from jax import lax
from jax.experimental import pallas as pl
from jax.experimental.pallas import tpu as pltpu
from jax.experimental.pallas import tpu_sc as plsc


def _make_gather(V, D, B):
    info = plsc.get_sparse_core_info()
    NC, NS = info.num_cores, info.num_subcores
    NW = NC * NS
    assert B % NW == 0
    b_per_w = B // NW
    C = 64
    assert b_per_w % C == 0
    steps = b_per_w // C

    mesh = plsc.VectorSubcoreMesh(core_axis_name="c", subcore_axis_name="s")

    NBUF = 10
    assert steps % NBUF == 0
    rounds = steps // NBUF

    @functools.partial(
        pl.kernel,
        mesh=mesh,
        out_type=jax.ShapeDtypeStruct((B, D), jnp.float32),
        scratch_types=[
            pltpu.VMEM((b_per_w,), jnp.int32),
            pltpu.VMEM((NBUF, C, D), jnp.float32),
            [pltpu.SemaphoreType.DMA] * NBUF,
            [pltpu.SemaphoreType.DMA] * NBUF,
        ],
    )
    def k(table_hbm, idx_hbm, out_hbm, idx_v, rows, gsems, osems):
        wid = lax.axis_index("s") * NC + lax.axis_index("c")
        base = wid * b_per_w
        pltpu.sync_copy(idx_hbm.at[pl.ds(base, b_per_w)], idx_v)

        def gather(g, b):
            return pltpu.make_async_copy(
                table_hbm.at[idx_v.at[pl.ds(g * C, C)]], rows.at[b], gsems[b]
            )

        def put(g, b):
            return pltpu.make_async_copy(
                rows.at[b], out_hbm.at[pl.ds(base + g * C, C)], osems[b]
            )

        for b in range(NBUF):
            gather(b, b).start()

        def step(i, carry):
            g0 = i * NBUF
            for b in range(NBUF):
                gather(g0 + b, b).wait()
                put(g0 + b, b).start()
            for b in range(NBUF):
                put(g0 + b, b).wait()
                gather(g0 + NBUF + b, b).start()
            return carry

        lax.fori_loop(0, rounds - 1, step, 0)

        g0 = (rounds - 1) * NBUF
        for b in range(NBUF):
            gather(g0 + b, b).wait()
            put(g0 + b, b).start()
        for b in range(NBUF):
            put(g0 + b, b).wait()

    return k


_GATHER_CACHE = {}


def kernel(input_ids, word_embeddings):
    V, D = word_embeddings.shape
    shape = input_ids.shape
    B = 1
    for s in shape:
        B *= s
    key = (V, D, B)
    if key not in _GATHER_CACHE:
        _GATHER_CACHE[key] = _make_gather(V, D, B)
    flat = input_ids.reshape(B).astype(jnp.int32)
    out = _GATHER_CACHE[key](word_embeddings, flat)
    return out.reshape(shape + (D,))


# speedup vs baseline: 1.0225x; 1.0083x over previous
import functools

import jax
import jax.numpy as jnp
from jax import lax
from jax.experimental import pallas as pl
from jax.experimental.pallas import tpu as pltpu
from jax.experimental.pallas import tpu_sc as plsc


def _make_gather(V, D, B):
    info = plsc.get_sparse_core_info()
    NC, NS = info.num_cores, info.num_subcores
    NW = NC * NS
    assert B % NW == 0
    b_per_w = B // NW
    C = 64
    assert b_per_w % C == 0
    steps = b_per_w // C

    mesh = plsc.VectorSubcoreMesh(core_axis_name="c", subcore_axis_name="s")

    NBUF = 10
    assert steps % NBUF == 0
    rounds = steps // NBUF

    HALF = NBUF // 2

    @functools.partial(
        pl.kernel,
        mesh=mesh,
        out_type=jax.ShapeDtypeStruct((B, D), jnp.float32),
        scratch_types=[
            pltpu.VMEM((b_per_w,), jnp.int32),
            pltpu.VMEM((NBUF * C, D), jnp.float32),
            [pltpu.SemaphoreType.DMA] * NBUF,
            pltpu.SemaphoreType.DMA,
            pltpu.SemaphoreType.DMA,
        ],
    )
    def k(table_hbm, idx_hbm, out_hbm, idx_v, rows, gsems, osem0, osem1):
        wid = lax.axis_index("s") * NC + lax.axis_index("c")
        base = wid * b_per_w
        pltpu.sync_copy(idx_hbm.at[pl.ds(base, b_per_w)], idx_v)

        def gather(g, b):
            return pltpu.make_async_copy(
                table_hbm.at[idx_v.at[pl.ds(g * C, C)]],
                rows.at[pl.ds(b * C, C)],
                gsems[b],
            )

        def half_put(g0, h, sem):
            return pltpu.make_async_copy(
                rows.at[pl.ds(h * HALF * C, HALF * C)],
                out_hbm.at[pl.ds(base + (g0 + h * HALF) * C, HALF * C)],
                sem,
            )

        for b in range(NBUF):
            gather(b, b).start()

        def step(i, carry):
            g0 = i * NBUF
            for b in range(HALF):
                gather(g0 + b, b).wait()
            half_put(g0, 0, osem0).start()
            for b in range(HALF, NBUF):
                gather(g0 + b, b).wait()
            half_put(g0, 1, osem1).start()
            half_put(g0, 0, osem0).wait()
            for b in range(HALF):
                gather(g0 + NBUF + b, b).start()
            half_put(g0, 1, osem1).wait()
            for b in range(HALF, NBUF):
                gather(g0 + NBUF + b, b).start()
            return carry

        lax.fori_loop(0, rounds - 1, step, 0)

        g0 = (rounds - 1) * NBUF
        for b in range(HALF):
            gather(g0 + b, b).wait()
        half_put(g0, 0, osem0).start()
        for b in range(HALF, NBUF):
            gather(g0 + b, b).wait()
        half_put(g0, 1, osem1).start()
        half_put(g0, 0, osem0).wait()
        half_put(g0, 1, osem1).wait()

    return k


_GATHER_CACHE = {}


def kernel(input_ids, word_embeddings):
    V, D = word_embeddings.shape
    shape = input_ids.shape
    B = 1
    for s in shape:
        B *= s
    key = (V, D, B)
    if key not in _GATHER_CACHE:
        _GATHER_CACHE[key] = _make_gather(V, D, B)
    flat = input_ids.reshape(B).astype(jnp.int32)
    out = _GATHER_CACHE[key](word_embeddings, flat)
    return out.reshape(shape + (D,))
